# baseline (speedup 1.0000x reference)
import numpy as np
import jax
import jax.numpy as jnp
from jax.experimental import pallas as pl

_GROUPING = np.array(
    [[0, 3, 6, 9], [1, 4, 7, 10], [2, 5, 8, 11],
     [12, 13, 14, 15], [16, 18, 20, 22], [17, 19, 21, 23]], dtype=np.int32)
_N_NODES = 24
_IN = 32
_WID = 32

_flat = _GROUPING.reshape(-1)
_inv = np.argsort(_flat)
_group_of = np.empty(_N_NODES, np.int64)
for _n in range(6):
    for _k in range(4):
        _group_of[_GROUPING[_n, _k]] = _n
_MASK_MAT = (_group_of[:, None] == _group_of[None, :]).astype(np.float32)

_BS = 4096


def _body(h_ref, v_ref, w1_ref, b1_ref, w2_ref, b2_ref, m_ref, o_ref):
    o = None
    for c in range(3):
        xc = jnp.maximum(h_ref[pl.ds(c * 256, 256), :],
                         0.0).astype(jnp.bfloat16)
        y = jnp.dot(w1_ref[c], xc,
                    preferred_element_type=jnp.float32)
        y = y + b1_ref[c]
        y = jnp.maximum(y, 0.0).astype(jnp.bfloat16)
        part = jnp.dot(w2_ref[c], y,
                       preferred_element_type=jnp.float32)
        o = part if o is None else o + part
    o = o + b2_ref[:]
    vf = v_ref[:].astype(jnp.float32)
    gs = jnp.dot(m_ref[:], vf, preferred_element_type=jnp.float32)
    o_ref[:, 0, :] = jnp.where(gs > 0.0, o, 0.0)


def kernel(h, valid, W1, b1, W2, b2):
    N = h.shape[0]
    inv = _inv
    W1n = W1.reshape(_N_NODES, _IN, _WID)[inv]
    W1t = W1n.transpose(0, 2, 1)
    eye8 = jnp.eye(8, dtype=W1.dtype)
    bd = (W1t.reshape(3, 8, _WID, _IN)[:, :, :, None, :]
          * eye8[None, :, None, :, None]).reshape(3, 256, 256)
    b1n = b1.reshape(_N_NODES, _WID)[inv].reshape(3, 256, 1)
    W2n = W2.reshape(_N_NODES, _WID)[inv]
    eye24 = jnp.eye(_N_NODES, dtype=W2.dtype)
    w2big = (eye24[:, None, :] * W2n[:, :, None]).reshape(768, _N_NODES)
    w2c = w2big.T.reshape(_N_NODES, 3, 256).transpose(1, 0, 2)
    b2n = b2.reshape(_N_NODES)[inv].reshape(_N_NODES, 1)
    Mm = jnp.asarray(_MASK_MAT)

    h2 = jnp.transpose(h, (1, 2, 0)).reshape(768, N)
    v2 = jnp.transpose(valid, (1, 0))

    grid = N // _BS
    out = pl.pallas_call(
        _body,
        grid=(grid,),
        in_specs=[
            pl.BlockSpec((768, _BS), lambda i: (0, i)),
            pl.BlockSpec((_N_NODES, _BS), lambda i: (0, i)),
            pl.BlockSpec((3, 256, 256), lambda i: (0, 0, 0)),
            pl.BlockSpec((3, 256, 1), lambda i: (0, 0, 0)),
            pl.BlockSpec((3, _N_NODES, 256), lambda i: (0, 0, 0)),
            pl.BlockSpec((_N_NODES, 1), lambda i: (0, 0)),
            pl.BlockSpec((_N_NODES, _N_NODES), lambda i: (0, 0)),
        ],
        out_specs=pl.BlockSpec((_N_NODES, 1, _BS), lambda i: (0, 0, i)),
        out_shape=jax.ShapeDtypeStruct((_N_NODES, 1, N), jnp.float32),
    )(h2, v2, bd.astype(jnp.bfloat16), b1n,
      w2c.astype(jnp.bfloat16), b2n, Mm)
    return jnp.transpose(out, (2, 0, 1))


# speedup vs baseline: 1.0133x; 1.0133x over previous
import numpy as np
import jax
import jax.numpy as jnp
from jax.experimental import pallas as pl

_GROUPING = np.array(
    [[0, 3, 6, 9], [1, 4, 7, 10], [2, 5, 8, 11],
     [12, 13, 14, 15], [16, 18, 20, 22], [17, 19, 21, 23]], dtype=np.int32)
_N_NODES = 24
_IN = 32
_WID = 32

_flat = _GROUPING.reshape(-1)
_inv = np.argsort(_flat)
_group_of = np.empty(_N_NODES, np.int64)
for _n in range(6):
    for _k in range(4):
        _group_of[_GROUPING[_n, _k]] = _n
_MASK_MAT = (_group_of[:, None] == _group_of[None, :]).astype(np.float32)

_BS = 4096


def _body(h_ref, v_ref, w1_ref, b1_ref, w2_ref, b2_ref, m_ref, o_ref):
    x = jnp.maximum(h_ref[:], 0.0).astype(jnp.bfloat16)
    ys = []
    for c in range(3):
        y = jnp.dot(w1_ref[c], x[c * 256:(c + 1) * 256, :],
                    preferred_element_type=jnp.float32)
        y = y + b1_ref[c]
        ys.append(jnp.maximum(y, 0.0).astype(jnp.bfloat16))
    yc = jnp.concatenate(ys, axis=0)
    o = jnp.dot(w2_ref[:], yc, preferred_element_type=jnp.float32)
    o = o + b2_ref[:]
    vf = v_ref[:].astype(jnp.float32)
    gs = jnp.dot(m_ref[:], vf, preferred_element_type=jnp.float32)
    o_ref[:, 0, :] = jnp.where(gs > 0.0, o, 0.0)


def kernel(h, valid, W1, b1, W2, b2):
    N = h.shape[0]
    inv = _inv
    W1n = W1.reshape(_N_NODES, _IN, _WID)[inv]
    W1t = W1n.transpose(0, 2, 1)
    eye8 = jnp.eye(8, dtype=W1.dtype)
    bd = (W1t.reshape(3, 8, _WID, _IN)[:, :, :, None, :]
          * eye8[None, :, None, :, None]).reshape(3, 256, 256)
    b1n = b1.reshape(_N_NODES, _WID)[inv].reshape(3, 256, 1)
    W2n = W2.reshape(_N_NODES, _WID)[inv]
    eye24 = jnp.eye(_N_NODES, dtype=W2.dtype)
    w2big = (eye24[:, None, :] * W2n[:, :, None]).reshape(768, _N_NODES)
    w2t = w2big.T
    b2n = b2.reshape(_N_NODES)[inv].reshape(_N_NODES, 1)
    Mm = jnp.asarray(_MASK_MAT)

    h2 = jnp.transpose(h, (1, 2, 0)).reshape(768, N)
    v2 = jnp.transpose(valid, (1, 0))

    grid = N // _BS
    out = pl.pallas_call(
        _body,
        grid=(grid,),
        in_specs=[
            pl.BlockSpec((768, _BS), lambda i: (0, i)),
            pl.BlockSpec((_N_NODES, _BS), lambda i: (0, i)),
            pl.BlockSpec((3, 256, 256), lambda i: (0, 0, 0)),
            pl.BlockSpec((3, 256, 1), lambda i: (0, 0, 0)),
            pl.BlockSpec((_N_NODES, 768), lambda i: (0, 0)),
            pl.BlockSpec((_N_NODES, 1), lambda i: (0, 0)),
            pl.BlockSpec((_N_NODES, _N_NODES), lambda i: (0, 0)),
        ],
        out_specs=pl.BlockSpec((_N_NODES, 1, _BS), lambda i: (0, 0, i)),
        out_shape=jax.ShapeDtypeStruct((_N_NODES, 1, N), jnp.float32),
    )(h2, v2, bd.astype(jnp.bfloat16), b1n,
      w2t.astype(jnp.bfloat16), b2n, Mm)
    return jnp.transpose(out, (2, 0, 1))
